# Initial kernel scaffold; baseline (speedup 1.0000x reference)
import jax
import jax.numpy as jnp
from jax.experimental import pallas as pl


def kernel(x, W_route, conv_w0, conv_w1, W_gate, ln_gamma, ln_beta):
    raise NotImplementedError("write your pallas kernel here")


# speedup vs baseline: 1.7428x; 1.7428x over previous
import functools

import jax
import jax.numpy as jnp
from jax.experimental import pallas as pl
from jax.experimental.pallas import tpu as pltpu

B, L, D = 4, 8192, 768
BUCKET = 8
PAD = BUCKET // 2


def _finale_body(x_ref, routed_ref, w1t_ref, w2t_ref, ksum_ref, gamma_ref,
                 beta_ref, out_ref):
    x = x_ref[...]
    routed = routed_ref[...] * 0.5
    logits = jnp.dot(x, w1t_ref[...], preferred_element_type=jnp.float32)
    logits += jnp.dot(routed, w2t_ref[...], preferred_element_type=jnp.float32)
    gate = jax.nn.sigmoid(logits + ksum_ref[...])
    y = x + gate * routed
    mean = jnp.mean(y, axis=-1, keepdims=True)
    yc = y - mean
    var = jnp.mean(yc * yc, axis=-1, keepdims=True)
    out_ref[...] = gamma_ref[...] * yc * jax.lax.rsqrt(var + 1e-5) + beta_ref[...]


def _finale(x2d, routed2d, w1t, w2t, ksum, gamma, beta):
    n = x2d.shape[0]
    blk = 1024
    grid = (n // blk,)
    return pl.pallas_call(
        _finale_body,
        grid=grid,
        in_specs=[
            pl.BlockSpec((blk, D), lambda i: (i, 0)),
            pl.BlockSpec((blk, D), lambda i: (i, 0)),
            pl.BlockSpec((D, D), lambda i: (0, 0)),
            pl.BlockSpec((D, D), lambda i: (0, 0)),
            pl.BlockSpec((blk, 1), lambda i: (i, 0)),
            pl.BlockSpec((1, D), lambda i: (0, 0)),
            pl.BlockSpec((1, D), lambda i: (0, 0)),
        ],
        out_specs=pl.BlockSpec((blk, D), lambda i: (i, 0)),
        out_shape=jax.ShapeDtypeStruct((n, D), jnp.float32),
    )(x2d, routed2d, w1t, w2t, ksum, gamma, beta)


def _route_and_exchange(x, keys_r, conv_w):
    sort_idx = jnp.argsort(keys_r, axis=1)
    unsort_idx = jnp.argsort(sort_idx, axis=1)
    x_sorted = jnp.take_along_axis(x, sort_idx[:, :, None], axis=1)
    h = jnp.transpose(x_sorted, (0, 2, 1))
    h_ex = jax.lax.conv_general_dilated(
        h, conv_w, window_strides=(1,), padding=[(PAD, PAD)],
        feature_group_count=D, dimension_numbers=("NCH", "OIH", "NCH"))
    h_ex = jnp.transpose(h_ex, (0, 2, 1))
    return jnp.take_along_axis(h_ex, unsort_idx[:, :, None], axis=1)


def kernel(x, W_route, conv_w0, conv_w1, W_gate, ln_gamma, ln_beta):
    route_keys = x @ W_route.T
    routed2 = _route_and_exchange(x, route_keys[:, :, 0], conv_w0) \
            + _route_and_exchange(x, route_keys[:, :, 1], conv_w1)
    ksum = jnp.sum(route_keys, axis=-1, keepdims=True)
    out = _finale(
        x.reshape(B * L, D),
        routed2.reshape(B * L, D),
        W_gate[:, :D].T,
        W_gate[:, D:].T,
        ksum.reshape(B * L, 1),
        ln_gamma.reshape(1, D),
        ln_beta.reshape(1, D),
    )
    return out.reshape(B, L, D)
